# baseline (speedup 1.0000x reference)
import dataclasses
import functools

import jax
import jax.numpy as jnp
from jax import lax
from jax.experimental import pallas as pl
from jax.experimental.pallas import tpu as pltpu
from jax.experimental.pallas import tpu_sc as plsc

N = 10000
E = 160000
D = 256
DC = 512
D_OUT = 64
GAMMA = 1.0

N_PAD = 10240
ROWS = 1024
GRID = N_PAD // ROWS

ALPHA = 0.25
BETA = float((0.5 / 256.0) ** 0.5)

NW = 32
E_PAD = 163840
EPW = E_PAD // NW
B_E = 64
NBLK_E = EPW // B_E
B_A = 80
FCH = 128
NSL = N_PAD // 16

_MESH = plsc.VectorSubcoreMesh(core_axis_name="c", subcore_axis_name="s")

_SC_PARAMS = pltpu.CompilerParams()
if "needs_layout_passes" in pltpu.CompilerParams.__dataclass_fields__:
    _SC_PARAMS = dataclasses.replace(_SC_PARAMS, needs_layout_passes=False)


def _stage1_body(h_ref, wfc_ref, wmu_ref, bmu_ref, wls_ref, bls_ref,
                 hflo_ref, hfhi_ref, c_ref):
    h = h_ref[...]
    hf = jnp.dot(h, wfc_ref[...], preferred_element_type=jnp.float32)
    hflo_ref[...] = hf[:, :FCH]
    hfhi_ref[...] = hf[:, FCH:]
    mu = jnp.dot(h, wmu_ref[...], preferred_element_type=jnp.float32) + bmu_ref[...]
    sg = jnp.exp(
        jnp.dot(h, wls_ref[...], preferred_element_type=jnp.float32) + bls_ref[...])
    c_ref[:, :D] = (mu * ALPHA).astype(jnp.bfloat16)
    c_ref[:, D:] = (sg * BETA).astype(jnp.bfloat16)


def _stage1(h_pad, W_fc, W_mu, b_mu, W_ls, b_ls):
    row_spec = pl.BlockSpec((ROWS, D), lambda i: (i, 0))
    half_spec = pl.BlockSpec((ROWS, FCH), lambda i: (i, 0))
    w_spec = pl.BlockSpec((D, D), lambda i: (0, 0))
    b_spec = pl.BlockSpec((1, D), lambda i: (0, 0))
    half = jax.ShapeDtypeStruct((N_PAD, FCH), jnp.float32)
    return pl.pallas_call(
        _stage1_body,
        grid=(GRID,),
        in_specs=[row_spec, w_spec, w_spec, b_spec, w_spec, b_spec],
        out_specs=[half_spec, half_spec, pl.BlockSpec((ROWS, DC), lambda i: (i, 0))],
        out_shape=[half, half, jax.ShapeDtypeStruct((N_PAD, DC), jnp.bfloat16)],
    )(h_pad, W_fc, W_mu, b_mu.reshape(1, D), W_ls, b_ls.reshape(1, D))


def _combine_body(lo0_ref, lo1_ref, hi0_ref, hi1_ref, hflo_ref, hfhi_ref, o_ref):
    o_ref[:, :FCH] = jnp.tanh(lo0_ref[...] + lo1_ref[...]
                              + GAMMA * hflo_ref[...])
    o_ref[:, FCH:] = jnp.tanh(hi0_ref[...] + hi1_ref[...]
                              + GAMMA * hfhi_ref[...])


def _combine(agg_lo, agg_hi, hf_lo, hf_hi):
    f_spec0 = pl.BlockSpec((ROWS, FCH), lambda i: (i, 0))
    f_spec1 = pl.BlockSpec((ROWS, FCH), lambda i: (i + GRID, 0))
    return pl.pallas_call(
        _combine_body,
        grid=(GRID,),
        in_specs=[f_spec0, f_spec1, f_spec0, f_spec1, f_spec0, f_spec0],
        out_specs=pl.BlockSpec((ROWS, D), lambda i: (i, 0)),
        out_shape=jax.ShapeDtypeStruct((N_PAD, D), jnp.float32),
    )(agg_lo, agg_lo, agg_hi, agg_hi, hf_lo, hf_hi)


def _head_body(x_ref, w_ref, o_ref):
    y = jnp.dot(x_ref[...], w_ref[...], preferred_element_type=jnp.float32)
    m = jnp.max(y, axis=-1, keepdims=True)
    ey = jnp.exp(y - m)
    o_ref[...] = ey / jnp.sum(ey, axis=-1, keepdims=True)


def _head(x_pad, W_out):
    return pl.pallas_call(
        _head_body,
        grid=(GRID,),
        in_specs=[pl.BlockSpec((ROWS, D), lambda i: (i, 0)),
                  pl.BlockSpec((D, D_OUT), lambda i: (0, 0))],
        out_specs=pl.BlockSpec((ROWS, D_OUT), lambda i: (i, 0)),
        out_shape=jax.ShapeDtypeStruct((N_PAD, D_OUT), jnp.float32),
    )(x_pad, W_out)


def _edot_block(cs, cd, ebig, off, mbuf):
    lanes = lax.iota(jnp.int32, 16)
    for g in range(B_E // 16):
        evec = jnp.zeros((16,), jnp.float32)
        for u in range(16):
            i = g * 16 + u
            acc = [jnp.zeros((16,), jnp.float32) for _ in range(4)]
            for j in range(DC // 32):
                va = plsc.bitcast(cs[i, pl.ds(16 * j, 16)], jnp.bfloat16)
                vb = plsc.bitcast(cd[i, pl.ds(16 * j, 16)], jnp.bfloat16)
                ps = va * vb
                x0, x1 = plsc.unpack(ps, format=plsc.PackFormat.INTERLEAVED)
                w = (2 * j) % 4
                acc[w] = acc[w] + x0
                acc[w + 1] = acc[w + 1] + x1
            tot = (acc[0] + acc[1]) + (acc[2] + acc[3])
            evec = jnp.where(lanes == u, jnp.sum(tot), evec)
        ebig[pl.ds(off + g * 16, 16)] = evec
        mbuf[...] = jnp.maximum(mbuf[...], evec)


def _epass(c_tab, src, dst):
    @functools.partial(
        pl.kernel,
        out_type=(jax.ShapeDtypeStruct((E_PAD,), jnp.float32),
                  jax.ShapeDtypeStruct((2 * (N_PAD // 128), 128), jnp.float32),
                  jax.ShapeDtypeStruct((2, 16), jnp.float32)),
        mesh=_MESH,
        compiler_params=_SC_PARAMS,
        scratch_types=[
            pltpu.VMEM((B_E,), jnp.int32),
            pltpu.VMEM((B_E,), jnp.int32),
            pltpu.VMEM((B_E,), jnp.int32),
            pltpu.VMEM((B_E,), jnp.int32),
            pltpu.VMEM((B_E, DC // 2), jnp.int32),
            pltpu.VMEM((B_E, DC // 2), jnp.int32),
            pltpu.VMEM((B_E, DC // 2), jnp.int32),
            pltpu.VMEM((B_E, DC // 2), jnp.int32),
            pltpu.VMEM((EPW,), jnp.float32),
            pltpu.VMEM((EPW,), jnp.int32),
            pltpu.VMEM((N_PAD // 128, 128), jnp.float32),
            pltpu.VMEM((N_PAD // 128,), jnp.int32),
            pltpu.VMEM((16,), jnp.float32),
            pltpu.VMEM((256,), jnp.float32),
            pltpu.VMEM_SHARED((N_PAD // 128, 128), jnp.float32),
            pltpu.VMEM_SHARED((256,), jnp.float32),
            pltpu.SemaphoreType.DMA,
            pltpu.SemaphoreType.DMA,
            pltpu.SemaphoreType.DMA,
            pltpu.SemaphoreType.DMA,
        ],
    )
    def k(c_hbm, src_hbm, dst_hbm, e_hbm, t_hbm, m_hbm,
          sidx0, didx0, sidx1, didx1, cs0, cd0, cs1, cd1,
          ebig, dbig, tbuf, ridx, mbuf, mall, tsh, msh,
          isem0, isem1, gsem0, gsem1):
        cid = lax.axis_index("c")
        sid = lax.axis_index("s")
        wid = cid * 16 + sid
        base = wid * EPW
        lanes = lax.iota(jnp.int32, 16)
        zero = jnp.zeros((16,), jnp.float32)

        mbuf[...] = zero - 3e38

        def idx_copy(b, sidx, didx, isem):
            pltpu.async_copy(src_hbm.at[pl.ds(base + b * B_E, B_E)], sidx, isem)
            pltpu.async_copy(dst_hbm.at[pl.ds(base + b * B_E, B_E)], didx, isem)

        def idx_wait(sidx, didx, isem):
            pltpu.make_async_copy(src_hbm.at[pl.ds(base, B_E)], sidx, isem).wait()
            pltpu.make_async_copy(dst_hbm.at[pl.ds(base, B_E)], didx, isem).wait()

        def gather(sidx, didx, cs, cd, gsem):
            pltpu.async_copy(c_hbm.at[sidx], cs, gsem)
            pltpu.async_copy(c_hbm.at[didx], cd, gsem)

        def gather_wait(cs, cd, gsem):
            pltpu.make_async_copy(c_hbm.at[pl.ds(0, B_E)], cs, gsem).wait()
            pltpu.make_async_copy(c_hbm.at[pl.ds(0, B_E)], cd, gsem).wait()

        idx_copy(0, sidx0, didx0, isem0)
        idx_wait(sidx0, didx0, isem0)
        gather(sidx0, didx0, cs0, cd0, gsem0)
        idx_copy(1, sidx1, didx1, isem1)

        @pl.loop(0, NBLK_E // 2)
        def _(t):
            b0 = 2 * t
            not_last = t < NBLK_E // 2 - 1

            idx_wait(sidx1, didx1, isem1)
            gather(sidx1, didx1, cs1, cd1, gsem1)
            gather_wait(cs0, cd0, gsem0)

            @pl.when(not_last)
            def _():
                idx_copy(b0 + 2, sidx0, didx0, isem0)
            _edot_block(cs0, cd0, ebig, b0 * B_E, mbuf)

            @pl.when(not_last)
            def _():
                idx_wait(sidx0, didx0, isem0)
                gather(sidx0, didx0, cs0, cd0, gsem0)
            gather_wait(cs1, cd1, gsem1)

            @pl.when(not_last)
            def _():
                idx_copy(b0 + 3, sidx1, didx1, isem1)
            _edot_block(cs1, cd1, ebig, (b0 + 1) * B_E, mbuf)

        @pl.loop(0, N_PAD // 128)
        def _(i):
            for j in range(8):
                tbuf[i, pl.ds(j * 16, 16)] = zero

        @pl.loop(0, N_PAD // 128 // 16)
        def _(i):
            ridx[pl.ds(i * 16, 16)] = lanes + i * 16

        pltpu.sync_copy(mbuf, msh.at[pl.ds(sid * 16, 16)])

        @pl.when(sid == 0)
        def _():
            pltpu.sync_copy(tbuf, tsh)
        plsc.subcore_barrier()

        pltpu.sync_copy(msh, mall)
        mv = jnp.maximum(mall[pl.ds(0, 16)], mall[pl.ds(16, 16)])
        for kk in range(2, 16):
            mv = jnp.maximum(mv, mall[pl.ds(kk * 16, 16)])
        mcv = jnp.where(lanes >= 0, jnp.max(mv), mv)

        pltpu.sync_copy(dst_hbm.at[pl.ds(base, EPW)], dbig)

        @pl.loop(0, EPW // 16)
        def _(i):
            dv = dbig[pl.ds(i * 16, 16)]
            ex = jnp.exp(ebig[pl.ds(i * 16, 16)] - mcv)
            plsc.addupdate_scatter(
                tbuf, [lax.shift_right_logical(dv, 7), lax.bitwise_and(dv, 127)], ex)

        pltpu.sync_copy(ebig, e_hbm.at[pl.ds(base, EPW)])
        pltpu.sync_copy(tbuf, tsh.at[ridx], add=True)

        @pl.when(sid == 0)
        def _():
            mbuf[...] = mcv
            pltpu.sync_copy(mbuf, m_hbm.at[cid])
        plsc.subcore_barrier()

        @pl.when(sid < 10)
        def _():
            pltpu.sync_copy(tsh.at[pl.ds(sid * 8, 8)],
                            t_hbm.at[pl.ds(cid * 80 + sid * 8, 8)])

    return k(c_tab, src, dst)


def _aggpass(hf_lo, hf_hi, src, dst, e, t_flat, m2, zslice):
    out_t = jax.ShapeDtypeStruct((2 * N_PAD, FCH), jnp.float32)

    @functools.partial(
        pl.kernel,
        out_type=(out_t, out_t),
        mesh=_MESH,
        compiler_params=_SC_PARAMS,
        scratch_types=[
            pltpu.VMEM((N_PAD // 128, 128), jnp.float32),
            pltpu.VMEM((N_PAD // 128, 128), jnp.float32),
            pltpu.VMEM((16,), jnp.float32),
            pltpu.VMEM((16,), jnp.float32),
            pltpu.VMEM((B_A,), jnp.int32),
            pltpu.VMEM((B_A,), jnp.int32),
            pltpu.VMEM((B_A,), jnp.float32),
            pltpu.VMEM((B_A,), jnp.float32),
            pltpu.VMEM((B_A,), jnp.int32),
            pltpu.VMEM((B_A,), jnp.int32),
            pltpu.VMEM((B_A,), jnp.float32),
            pltpu.VMEM((B_A,), jnp.float32),
            pltpu.VMEM((B_A, FCH), jnp.float32),
            pltpu.VMEM((B_A, FCH), jnp.float32),
            pltpu.VMEM_SHARED((N_PAD, FCH), jnp.float32),
            pltpu.SemaphoreType.DMA,
            pltpu.SemaphoreType.DMA,
            pltpu.SemaphoreType.DMA,
            pltpu.SemaphoreType.DMA,
            pltpu.SemaphoreType.DMA,
            pltpu.SemaphoreType.DMA,
        ],
    )
    def k(hf_lo_hbm, hf_hi_hbm, src_hbm, dst_hbm, e_hbm, t_hbm, m_hbm, z_hbm,
          out_lo_hbm, out_hi_hbm, ttab0, ttab1, mb0, mb1,
          sidx0, didx0, ebuf0, abuf0, sidx1, didx1, ebuf1, abuf1,
          rows0, rows1, acc, isem0, isem1, gsem0, gsem1, ssem0, ssem1):
        cid = lax.axis_index("c")
        sid = lax.axis_index("s")
        wid = cid * 16 + sid
        base = wid * EPW
        my_row0 = sid * NSL
        nblk = EPW // B_A

        pltpu.sync_copy(t_hbm.at[pl.ds(0, N_PAD // 128)], ttab0)
        pltpu.sync_copy(t_hbm.at[pl.ds(N_PAD // 128, N_PAD // 128)], ttab1)
        pltpu.sync_copy(m_hbm.at[0], mb0)
        pltpu.sync_copy(m_hbm.at[1], mb1)
        m0 = mb0[...]
        m1 = mb1[...]
        mx = jnp.maximum(m0, m1)
        w0 = jnp.exp(m0 - mx)
        w1 = jnp.exp(m1 - mx)

        def idx_copy(b, sidx, didx, ebuf, isem):
            pltpu.async_copy(src_hbm.at[pl.ds(base + b * B_A, B_A)], sidx, isem)
            pltpu.async_copy(dst_hbm.at[pl.ds(base + b * B_A, B_A)], didx, isem)
            pltpu.async_copy(e_hbm.at[pl.ds(base + b * B_A, B_A)], ebuf, isem)

        def idx_wait(sidx, didx, ebuf, isem):
            pltpu.make_async_copy(src_hbm.at[pl.ds(base, B_A)], sidx, isem).wait()
            pltpu.make_async_copy(dst_hbm.at[pl.ds(base, B_A)], didx, isem).wait()
            pltpu.make_async_copy(e_hbm.at[pl.ds(base, B_A)], ebuf, isem).wait()

        def scale(hf_hbm, rows, didx, ebuf, abuf, gsem):
            pltpu.make_async_copy(hf_hbm.at[pl.ds(0, B_A)], rows, gsem).wait()

            @pl.loop(0, B_A // 16)
            def _(q):
                dv = didx[pl.ds(q * 16, 16)]
                dhi = lax.shift_right_logical(dv, 7)
                dlo = lax.bitwise_and(dv, 127)
                t0 = plsc.load_gather(ttab0, [dhi, dlo])
                t1 = plsc.load_gather(ttab1, [dhi, dlo])
                den = jnp.maximum(t0 * w0 + t1 * w1, 1e-37)
                abuf[pl.ds(q * 16, 16)] = (
                    jnp.exp(ebuf[pl.ds(q * 16, 16)] - mx) / den)

            @pl.loop(0, B_A)
            def _(i):
                av = plsc.load_gather(abuf, [jnp.full((16,), i, jnp.int32)])
                for j in range(FCH // 16):
                    sl = pl.ds(j * 16, 16)
                    rows[i, sl] = rows[i, sl] * av

        def scat_wait(rows, ssem):
            pltpu.make_async_copy(rows, acc.at[pl.ds(0, B_A)], ssem).wait()

        for hf_hbm, out_hbm in ((hf_lo_hbm, out_lo_hbm), (hf_hi_hbm, out_hi_hbm)):
            pltpu.sync_copy(z_hbm, acc.at[pl.ds(my_row0, NSL)])
            plsc.subcore_barrier()

            idx_copy(0, sidx0, didx0, ebuf0, isem0)
            idx_wait(sidx0, didx0, ebuf0, isem0)
            pltpu.async_copy(hf_hbm.at[sidx0], rows0, gsem0)
            idx_copy(1, sidx1, didx1, ebuf1, isem1)

            @pl.loop(0, nblk // 2)
            def _(t):
                b0 = 2 * t
                not_last = t < nblk // 2 - 1

                idx_wait(sidx1, didx1, ebuf1, isem1)
                pltpu.async_copy(hf_hbm.at[sidx1], rows1, gsem1)
                scale(hf_hbm, rows0, didx0, ebuf0, abuf0, gsem0)
                pltpu.async_copy(rows0, acc.at[didx0], ssem0, add=True)

                scale(hf_hbm, rows1, didx1, ebuf1, abuf1, gsem1)
                pltpu.async_copy(rows1, acc.at[didx1], ssem1, add=True)

                @pl.when(not_last)
                def _():
                    scat_wait(rows0, ssem0)
                    idx_copy(b0 + 2, sidx0, didx0, ebuf0, isem0)
                    idx_wait(sidx0, didx0, ebuf0, isem0)
                    pltpu.async_copy(hf_hbm.at[sidx0], rows0, gsem0)
                    scat_wait(rows1, ssem1)
                    idx_copy(b0 + 3, sidx1, didx1, ebuf1, isem1)

            scat_wait(rows0, ssem0)
            scat_wait(rows1, ssem1)
            plsc.subcore_barrier()
            pltpu.sync_copy(acc.at[pl.ds(my_row0, NSL)],
                            out_hbm.at[pl.ds(cid * N_PAD + my_row0, NSL)])
            plsc.subcore_barrier()

    return k(hf_lo, hf_hi, src, dst, e, t_flat, m2, zslice)


def _layer(h_pad, src_p, dst_p, zslice, W_fc, W_mu, b_mu, W_ls, b_ls):
    hf_lo, hf_hi, c = _stage1(h_pad, W_fc, W_mu, b_mu, W_ls, b_ls)
    c_i32 = lax.bitcast_convert_type(c.reshape(N_PAD, DC // 2, 2), jnp.int32)
    e, t_flat, m2 = _epass(c_i32, src_p, dst_p)
    agg_lo, agg_hi = _aggpass(hf_lo, hf_hi, src_p, dst_p, e, t_flat, m2, zslice)
    return _combine(agg_lo, agg_hi, hf_lo, hf_hi)


def kernel(h, edge_index, W_fc0, W_mu0, b_mu0, W_ls0, b_ls0,
           W_fc1, W_mu1, b_mu1, W_ls1, b_ls1, W_out):
    src_p = jnp.concatenate([edge_index[0], jnp.zeros((E_PAD - E,), jnp.int32)])
    dst_p = jnp.concatenate(
        [edge_index[1], jnp.full((E_PAD - E,), N_PAD - 1, jnp.int32)])
    zslice = jnp.zeros((NSL, FCH), jnp.float32)
    h_pad = jnp.pad(h, ((0, N_PAD - N), (0, 0)))
    h1 = _layer(h_pad, src_p, dst_p, zslice, W_fc0, W_mu0, b_mu0, W_ls0, b_ls0)
    h2 = _layer(h1, src_p, dst_p, zslice, W_fc1, W_mu1, b_mu1, W_ls1, b_ls1)
    return _head(h2, W_out)[:N]


# speedup vs baseline: 1.0171x; 1.0171x over previous
import dataclasses
import functools

import jax
import jax.numpy as jnp
from jax import lax
from jax.experimental import pallas as pl
from jax.experimental.pallas import tpu as pltpu
from jax.experimental.pallas import tpu_sc as plsc

N = 10000
E = 160000
D = 256
DC = 512
D_OUT = 64
GAMMA = 1.0

N_PAD = 10240
ROWS = 1024
GRID = N_PAD // ROWS

ALPHA = 0.25
BETA = float((0.5 / 256.0) ** 0.5)

NW = 32
E_PAD = 163840
EPW = E_PAD // NW
B_E = 64
NBLK_E = EPW // B_E
B_A = 128
FCH = 128
NSL = N_PAD // 16

_MESH = plsc.VectorSubcoreMesh(core_axis_name="c", subcore_axis_name="s")

_SC_PARAMS = pltpu.CompilerParams()
if "needs_layout_passes" in pltpu.CompilerParams.__dataclass_fields__:
    _SC_PARAMS = dataclasses.replace(_SC_PARAMS, needs_layout_passes=False)


def _stage1_body(h_ref, wfc_ref, wmu_ref, bmu_ref, wls_ref, bls_ref,
                 hflo_ref, hfhi_ref, c_ref):
    h = h_ref[...]
    hf = jnp.dot(h, wfc_ref[...], preferred_element_type=jnp.float32)
    hflo_ref[...] = hf[:, :FCH]
    hfhi_ref[...] = hf[:, FCH:]
    mu = jnp.dot(h, wmu_ref[...], preferred_element_type=jnp.float32) + bmu_ref[...]
    sg = jnp.exp(
        jnp.dot(h, wls_ref[...], preferred_element_type=jnp.float32) + bls_ref[...])
    c_ref[:, :D] = (mu * ALPHA).astype(jnp.bfloat16)
    c_ref[:, D:] = (sg * BETA).astype(jnp.bfloat16)


def _stage1(h_pad, W_fc, W_mu, b_mu, W_ls, b_ls):
    row_spec = pl.BlockSpec((ROWS, D), lambda i: (i, 0))
    half_spec = pl.BlockSpec((ROWS, FCH), lambda i: (i, 0))
    w_spec = pl.BlockSpec((D, D), lambda i: (0, 0))
    b_spec = pl.BlockSpec((1, D), lambda i: (0, 0))
    half = jax.ShapeDtypeStruct((N_PAD, FCH), jnp.float32)
    return pl.pallas_call(
        _stage1_body,
        grid=(GRID,),
        in_specs=[row_spec, w_spec, w_spec, b_spec, w_spec, b_spec],
        out_specs=[half_spec, half_spec, pl.BlockSpec((ROWS, DC), lambda i: (i, 0))],
        out_shape=[half, half, jax.ShapeDtypeStruct((N_PAD, DC), jnp.bfloat16)],
    )(h_pad, W_fc, W_mu, b_mu.reshape(1, D), W_ls, b_ls.reshape(1, D))


def _combine_body(lo0_ref, lo1_ref, hi0_ref, hi1_ref, hflo_ref, hfhi_ref, o_ref):
    o_ref[:, :FCH] = jnp.tanh(lo0_ref[...] + lo1_ref[...]
                              + GAMMA * hflo_ref[...])
    o_ref[:, FCH:] = jnp.tanh(hi0_ref[...] + hi1_ref[...]
                              + GAMMA * hfhi_ref[...])


def _combine(agg_lo, agg_hi, hf_lo, hf_hi):
    f_spec0 = pl.BlockSpec((ROWS, FCH), lambda i: (i, 0))
    f_spec1 = pl.BlockSpec((ROWS, FCH), lambda i: (i + GRID, 0))
    return pl.pallas_call(
        _combine_body,
        grid=(GRID,),
        in_specs=[f_spec0, f_spec1, f_spec0, f_spec1, f_spec0, f_spec0],
        out_specs=pl.BlockSpec((ROWS, D), lambda i: (i, 0)),
        out_shape=jax.ShapeDtypeStruct((N_PAD, D), jnp.float32),
    )(agg_lo, agg_lo, agg_hi, agg_hi, hf_lo, hf_hi)


def _head_body(x_ref, w_ref, o_ref):
    y = jnp.dot(x_ref[...], w_ref[...], preferred_element_type=jnp.float32)
    m = jnp.max(y, axis=-1, keepdims=True)
    ey = jnp.exp(y - m)
    o_ref[...] = ey / jnp.sum(ey, axis=-1, keepdims=True)


def _head(x_pad, W_out):
    return pl.pallas_call(
        _head_body,
        grid=(GRID,),
        in_specs=[pl.BlockSpec((ROWS, D), lambda i: (i, 0)),
                  pl.BlockSpec((D, D_OUT), lambda i: (0, 0))],
        out_specs=pl.BlockSpec((ROWS, D_OUT), lambda i: (i, 0)),
        out_shape=jax.ShapeDtypeStruct((N_PAD, D_OUT), jnp.float32),
    )(x_pad, W_out)


def _edot_block(cs, cd, ebig, off, mbuf):
    lanes = lax.iota(jnp.int32, 16)
    for g in range(B_E // 16):
        evec = jnp.zeros((16,), jnp.float32)
        for u in range(16):
            i = g * 16 + u
            acc = [jnp.zeros((16,), jnp.float32) for _ in range(4)]
            for j in range(DC // 32):
                va = plsc.bitcast(cs[i, pl.ds(16 * j, 16)], jnp.bfloat16)
                vb = plsc.bitcast(cd[i, pl.ds(16 * j, 16)], jnp.bfloat16)
                ps = va * vb
                x0, x1 = plsc.unpack(ps, format=plsc.PackFormat.INTERLEAVED)
                w = (2 * j) % 4
                acc[w] = acc[w] + x0
                acc[w + 1] = acc[w + 1] + x1
            tot = (acc[0] + acc[1]) + (acc[2] + acc[3])
            evec = jnp.where(lanes == u, jnp.sum(tot), evec)
        ebig[pl.ds(off + g * 16, 16)] = evec
        mbuf[...] = jnp.maximum(mbuf[...], evec)


def _epass(c_tab, src, dst):
    @functools.partial(
        pl.kernel,
        out_type=(jax.ShapeDtypeStruct((E_PAD,), jnp.float32),
                  jax.ShapeDtypeStruct((2 * (N_PAD // 128), 128), jnp.float32),
                  jax.ShapeDtypeStruct((2, 16), jnp.float32)),
        mesh=_MESH,
        compiler_params=_SC_PARAMS,
        scratch_types=[
            pltpu.VMEM((B_E,), jnp.int32),
            pltpu.VMEM((B_E,), jnp.int32),
            pltpu.VMEM((B_E,), jnp.int32),
            pltpu.VMEM((B_E,), jnp.int32),
            pltpu.VMEM((B_E, DC // 2), jnp.int32),
            pltpu.VMEM((B_E, DC // 2), jnp.int32),
            pltpu.VMEM((B_E, DC // 2), jnp.int32),
            pltpu.VMEM((B_E, DC // 2), jnp.int32),
            pltpu.VMEM((EPW,), jnp.float32),
            pltpu.VMEM((EPW,), jnp.int32),
            pltpu.VMEM((N_PAD // 128, 128), jnp.float32),
            pltpu.VMEM((N_PAD // 128,), jnp.int32),
            pltpu.VMEM((16,), jnp.float32),
            pltpu.VMEM((256,), jnp.float32),
            pltpu.VMEM_SHARED((N_PAD // 128, 128), jnp.float32),
            pltpu.VMEM_SHARED((256,), jnp.float32),
            pltpu.SemaphoreType.DMA,
            pltpu.SemaphoreType.DMA,
            pltpu.SemaphoreType.DMA,
            pltpu.SemaphoreType.DMA,
        ],
    )
    def k(c_hbm, src_hbm, dst_hbm, e_hbm, t_hbm, m_hbm,
          sidx0, didx0, sidx1, didx1, cs0, cd0, cs1, cd1,
          ebig, dbig, tbuf, ridx, mbuf, mall, tsh, msh,
          isem0, isem1, gsem0, gsem1):
        cid = lax.axis_index("c")
        sid = lax.axis_index("s")
        wid = cid * 16 + sid
        base = wid * EPW
        lanes = lax.iota(jnp.int32, 16)
        zero = jnp.zeros((16,), jnp.float32)

        mbuf[...] = zero - 3e38

        def idx_copy(b, sidx, didx, isem):
            pltpu.async_copy(src_hbm.at[pl.ds(base + b * B_E, B_E)], sidx, isem)
            pltpu.async_copy(dst_hbm.at[pl.ds(base + b * B_E, B_E)], didx, isem)

        def idx_wait(sidx, didx, isem):
            pltpu.make_async_copy(src_hbm.at[pl.ds(base, B_E)], sidx, isem).wait()
            pltpu.make_async_copy(dst_hbm.at[pl.ds(base, B_E)], didx, isem).wait()

        def gather(sidx, didx, cs, cd, gsem):
            pltpu.async_copy(c_hbm.at[sidx], cs, gsem)
            pltpu.async_copy(c_hbm.at[didx], cd, gsem)

        def gather_wait(cs, cd, gsem):
            pltpu.make_async_copy(c_hbm.at[pl.ds(0, B_E)], cs, gsem).wait()
            pltpu.make_async_copy(c_hbm.at[pl.ds(0, B_E)], cd, gsem).wait()

        idx_copy(0, sidx0, didx0, isem0)
        idx_wait(sidx0, didx0, isem0)
        gather(sidx0, didx0, cs0, cd0, gsem0)
        idx_copy(1, sidx1, didx1, isem1)

        @pl.loop(0, NBLK_E // 2)
        def _(t):
            b0 = 2 * t
            not_last = t < NBLK_E // 2 - 1

            idx_wait(sidx1, didx1, isem1)
            gather(sidx1, didx1, cs1, cd1, gsem1)
            gather_wait(cs0, cd0, gsem0)

            @pl.when(not_last)
            def _():
                idx_copy(b0 + 2, sidx0, didx0, isem0)
            _edot_block(cs0, cd0, ebig, b0 * B_E, mbuf)

            @pl.when(not_last)
            def _():
                idx_wait(sidx0, didx0, isem0)
                gather(sidx0, didx0, cs0, cd0, gsem0)
            gather_wait(cs1, cd1, gsem1)

            @pl.when(not_last)
            def _():
                idx_copy(b0 + 3, sidx1, didx1, isem1)
            _edot_block(cs1, cd1, ebig, (b0 + 1) * B_E, mbuf)

        @pl.loop(0, N_PAD // 128)
        def _(i):
            for j in range(8):
                tbuf[i, pl.ds(j * 16, 16)] = zero

        @pl.loop(0, N_PAD // 128 // 16)
        def _(i):
            ridx[pl.ds(i * 16, 16)] = lanes + i * 16

        pltpu.sync_copy(mbuf, msh.at[pl.ds(sid * 16, 16)])

        @pl.when(sid == 0)
        def _():
            pltpu.sync_copy(tbuf, tsh)
        plsc.subcore_barrier()

        pltpu.sync_copy(msh, mall)
        mv = jnp.maximum(mall[pl.ds(0, 16)], mall[pl.ds(16, 16)])
        for kk in range(2, 16):
            mv = jnp.maximum(mv, mall[pl.ds(kk * 16, 16)])
        mcv = jnp.where(lanes >= 0, jnp.max(mv), mv)

        pltpu.sync_copy(dst_hbm.at[pl.ds(base, EPW)], dbig)

        @pl.loop(0, EPW // 16)
        def _(i):
            dv = dbig[pl.ds(i * 16, 16)]
            ex = jnp.exp(ebig[pl.ds(i * 16, 16)] - mcv)
            plsc.addupdate_scatter(
                tbuf, [lax.shift_right_logical(dv, 7), lax.bitwise_and(dv, 127)], ex)

        pltpu.sync_copy(ebig, e_hbm.at[pl.ds(base, EPW)])
        pltpu.sync_copy(tbuf, tsh.at[ridx], add=True)

        @pl.when(sid == 0)
        def _():
            mbuf[...] = mcv
            pltpu.sync_copy(mbuf, m_hbm.at[cid])
        plsc.subcore_barrier()

        @pl.when(sid < 10)
        def _():
            pltpu.sync_copy(tsh.at[pl.ds(sid * 8, 8)],
                            t_hbm.at[pl.ds(cid * 80 + sid * 8, 8)])

    return k(c_tab, src, dst)


def _aggpass(hf_lo, hf_hi, src, dst, e, t_flat, m2, zslice):
    out_t = jax.ShapeDtypeStruct((2 * N_PAD, FCH), jnp.float32)

    @functools.partial(
        pl.kernel,
        out_type=(out_t, out_t),
        mesh=_MESH,
        compiler_params=_SC_PARAMS,
        scratch_types=[
            pltpu.VMEM((N_PAD // 128, 128), jnp.float32),
            pltpu.VMEM((16,), jnp.float32),
            pltpu.VMEM((16,), jnp.float32),
            pltpu.VMEM((B_A,), jnp.int32),
            pltpu.VMEM((B_A,), jnp.int32),
            pltpu.VMEM((B_A,), jnp.float32),
            pltpu.VMEM((B_A,), jnp.float32),
            pltpu.VMEM((B_A,), jnp.int32),
            pltpu.VMEM((B_A,), jnp.int32),
            pltpu.VMEM((B_A,), jnp.float32),
            pltpu.VMEM((B_A,), jnp.float32),
            pltpu.VMEM((B_A, FCH), jnp.float32),
            pltpu.VMEM((B_A, FCH), jnp.float32),
            pltpu.VMEM_SHARED((N_PAD, FCH), jnp.float32),
            pltpu.SemaphoreType.DMA,
            pltpu.SemaphoreType.DMA,
            pltpu.SemaphoreType.DMA,
            pltpu.SemaphoreType.DMA,
            pltpu.SemaphoreType.DMA,
            pltpu.SemaphoreType.DMA,
        ],
    )
    def k(hf_lo_hbm, hf_hi_hbm, src_hbm, dst_hbm, e_hbm, t_hbm, m_hbm, z_hbm,
          out_lo_hbm, out_hi_hbm, rtab, mb0, mb1,
          sidx0, didx0, ebuf0, abuf0, sidx1, didx1, ebuf1, abuf1,
          rows0, rows1, acc, isem0, isem1, gsem0, gsem1, ssem0, ssem1):
        cid = lax.axis_index("c")
        sid = lax.axis_index("s")
        wid = cid * 16 + sid
        base = wid * EPW
        my_row0 = sid * NSL
        nblk = EPW // B_A

        pltpu.sync_copy(t_hbm.at[pl.ds(0, N_PAD // 128)], rtab)
        pltpu.sync_copy(t_hbm.at[pl.ds(N_PAD // 128, N_PAD // 128)],
                        rows0.at[pl.ds(0, N_PAD // 128)])
        pltpu.sync_copy(m_hbm.at[0], mb0)
        pltpu.sync_copy(m_hbm.at[1], mb1)
        m0 = mb0[...]
        m1 = mb1[...]
        mx = jnp.maximum(m0, m1)
        w0 = jnp.exp(m0 - mx)
        w1 = jnp.exp(m1 - mx)

        @pl.loop(0, N_PAD // 128)
        def _(r):
            for j in range(8):
                sl = pl.ds(j * 16, 16)
                den = jnp.maximum(rtab[r, sl] * w0 + rows0[r, sl] * w1, 1e-37)
                rtab[r, sl] = 1.0 / den

        def idx_copy(b, sidx, didx, ebuf, isem):
            pltpu.async_copy(src_hbm.at[pl.ds(base + b * B_A, B_A)], sidx, isem)
            pltpu.async_copy(dst_hbm.at[pl.ds(base + b * B_A, B_A)], didx, isem)
            pltpu.async_copy(e_hbm.at[pl.ds(base + b * B_A, B_A)], ebuf, isem)

        def idx_wait(sidx, didx, ebuf, isem):
            pltpu.make_async_copy(src_hbm.at[pl.ds(base, B_A)], sidx, isem).wait()
            pltpu.make_async_copy(dst_hbm.at[pl.ds(base, B_A)], didx, isem).wait()
            pltpu.make_async_copy(e_hbm.at[pl.ds(base, B_A)], ebuf, isem).wait()

        def scale(hf_hbm, rows, didx, ebuf, abuf, gsem):
            pltpu.make_async_copy(hf_hbm.at[pl.ds(0, B_A)], rows, gsem).wait()

            @pl.loop(0, B_A // 16)
            def _(q):
                dv = didx[pl.ds(q * 16, 16)]
                dhi = lax.shift_right_logical(dv, 7)
                dlo = lax.bitwise_and(dv, 127)
                rv = plsc.load_gather(rtab, [dhi, dlo])
                abuf[pl.ds(q * 16, 16)] = (
                    jnp.exp(ebuf[pl.ds(q * 16, 16)] - mx) * rv)

            @pl.loop(0, B_A)
            def _(i):
                av = plsc.load_gather(abuf, [jnp.full((16,), i, jnp.int32)])
                for j in range(FCH // 16):
                    sl = pl.ds(j * 16, 16)
                    rows[i, sl] = rows[i, sl] * av

        def scat_wait(rows, ssem):
            pltpu.make_async_copy(rows, acc.at[pl.ds(0, B_A)], ssem).wait()

        for hf_hbm, out_hbm in ((hf_lo_hbm, out_lo_hbm), (hf_hi_hbm, out_hi_hbm)):
            pltpu.sync_copy(z_hbm, acc.at[pl.ds(my_row0, NSL)])
            plsc.subcore_barrier()

            idx_copy(0, sidx0, didx0, ebuf0, isem0)
            idx_wait(sidx0, didx0, ebuf0, isem0)
            pltpu.async_copy(hf_hbm.at[sidx0], rows0, gsem0)
            idx_copy(1, sidx1, didx1, ebuf1, isem1)

            @pl.loop(0, nblk // 2)
            def _(t):
                b0 = 2 * t
                not_last = t < nblk // 2 - 1

                idx_wait(sidx1, didx1, ebuf1, isem1)
                pltpu.async_copy(hf_hbm.at[sidx1], rows1, gsem1)
                scale(hf_hbm, rows0, didx0, ebuf0, abuf0, gsem0)
                pltpu.async_copy(rows0, acc.at[didx0], ssem0, add=True)

                scale(hf_hbm, rows1, didx1, ebuf1, abuf1, gsem1)
                pltpu.async_copy(rows1, acc.at[didx1], ssem1, add=True)

                @pl.when(not_last)
                def _():
                    scat_wait(rows0, ssem0)
                    idx_copy(b0 + 2, sidx0, didx0, ebuf0, isem0)
                    idx_wait(sidx0, didx0, ebuf0, isem0)
                    pltpu.async_copy(hf_hbm.at[sidx0], rows0, gsem0)
                    scat_wait(rows1, ssem1)
                    idx_copy(b0 + 3, sidx1, didx1, ebuf1, isem1)

            scat_wait(rows0, ssem0)
            scat_wait(rows1, ssem1)
            plsc.subcore_barrier()
            pltpu.sync_copy(acc.at[pl.ds(my_row0, NSL)],
                            out_hbm.at[pl.ds(cid * N_PAD + my_row0, NSL)])
            plsc.subcore_barrier()

    return k(hf_lo, hf_hi, src, dst, e, t_flat, m2, zslice)


def _layer(h_pad, src_p, dst_p, zslice, W_fc, W_mu, b_mu, W_ls, b_ls):
    hf_lo, hf_hi, c = _stage1(h_pad, W_fc, W_mu, b_mu, W_ls, b_ls)
    c_i32 = lax.bitcast_convert_type(c.reshape(N_PAD, DC // 2, 2), jnp.int32)
    e, t_flat, m2 = _epass(c_i32, src_p, dst_p)
    agg_lo, agg_hi = _aggpass(hf_lo, hf_hi, src_p, dst_p, e, t_flat, m2, zslice)
    return _combine(agg_lo, agg_hi, hf_lo, hf_hi)


def kernel(h, edge_index, W_fc0, W_mu0, b_mu0, W_ls0, b_ls0,
           W_fc1, W_mu1, b_mu1, W_ls1, b_ls1, W_out):
    src_p = jnp.concatenate([edge_index[0], jnp.zeros((E_PAD - E,), jnp.int32)])
    dst_p = jnp.concatenate(
        [edge_index[1], jnp.full((E_PAD - E,), N_PAD - 1, jnp.int32)])
    zslice = jnp.zeros((NSL, FCH), jnp.float32)
    h_pad = jnp.pad(h, ((0, N_PAD - N), (0, 0)))
    h1 = _layer(h_pad, src_p, dst_p, zslice, W_fc0, W_mu0, b_mu0, W_ls0, b_ls0)
    h2 = _layer(h1, src_p, dst_p, zslice, W_fc1, W_mu1, b_mu1, W_ls1, b_ls1)
    return _head(h2, W_out)[:N]


# speedup vs baseline: 1.0206x; 1.0035x over previous
import dataclasses
import functools

import jax
import jax.numpy as jnp
from jax import lax
from jax.experimental import pallas as pl
from jax.experimental.pallas import tpu as pltpu
from jax.experimental.pallas import tpu_sc as plsc

N = 10000
E = 160000
D = 256
DC = 512
D_OUT = 64
GAMMA = 1.0

N_PAD = 10240
ROWS = 1024
GRID = N_PAD // ROWS

ALPHA = 0.25
BETA = float((0.5 / 256.0) ** 0.5)

NW = 32
E_PAD = 163840
EPW = E_PAD // NW
B_E = 64
NBLK_E = EPW // B_E
B_A = 128
FCH = 128
NSL = N_PAD // 16

_MESH = plsc.VectorSubcoreMesh(core_axis_name="c", subcore_axis_name="s")

_SC_PARAMS = pltpu.CompilerParams()
if "needs_layout_passes" in pltpu.CompilerParams.__dataclass_fields__:
    _SC_PARAMS = dataclasses.replace(_SC_PARAMS, needs_layout_passes=False)


def _stage1_body(h_ref, wfc_ref, wmu_ref, bmu_ref, wls_ref, bls_ref,
                 hflo_ref, hfhi_ref, c_ref):
    h = h_ref[...]
    hf = jnp.dot(h, wfc_ref[...], preferred_element_type=jnp.float32)
    hflo_ref[...] = hf[:, :FCH]
    hfhi_ref[...] = hf[:, FCH:]
    mu = jnp.dot(h, wmu_ref[...], preferred_element_type=jnp.float32) + bmu_ref[...]
    sg = jnp.exp(
        jnp.dot(h, wls_ref[...], preferred_element_type=jnp.float32) + bls_ref[...])
    c_ref[:, :D] = (mu * ALPHA).astype(jnp.bfloat16)
    c_ref[:, D:] = (sg * BETA).astype(jnp.bfloat16)


def _stage1(h_pad, W_fc, W_mu, b_mu, W_ls, b_ls):
    row_spec = pl.BlockSpec((ROWS, D), lambda i: (i, 0))
    half_spec = pl.BlockSpec((ROWS, FCH), lambda i: (i, 0))
    w_spec = pl.BlockSpec((D, D), lambda i: (0, 0))
    b_spec = pl.BlockSpec((1, D), lambda i: (0, 0))
    half = jax.ShapeDtypeStruct((N_PAD, FCH), jnp.float32)
    return pl.pallas_call(
        _stage1_body,
        grid=(GRID,),
        in_specs=[row_spec, w_spec, w_spec, b_spec, w_spec, b_spec],
        out_specs=[half_spec, half_spec, pl.BlockSpec((ROWS, DC), lambda i: (i, 0))],
        out_shape=[half, half, jax.ShapeDtypeStruct((N_PAD, DC), jnp.bfloat16)],
    )(h_pad, W_fc, W_mu, b_mu.reshape(1, D), W_ls, b_ls.reshape(1, D))


def _combine_body(lo0_ref, lo1_ref, hi0_ref, hi1_ref, hflo_ref, hfhi_ref, o_ref):
    o_ref[:, :FCH] = jnp.tanh(lo0_ref[...] + lo1_ref[...]
                              + GAMMA * hflo_ref[...])
    o_ref[:, FCH:] = jnp.tanh(hi0_ref[...] + hi1_ref[...]
                              + GAMMA * hfhi_ref[...])


def _combine(agg_lo, agg_hi, hf_lo, hf_hi):
    f_spec0 = pl.BlockSpec((ROWS, FCH), lambda i: (i, 0))
    f_spec1 = pl.BlockSpec((ROWS, FCH), lambda i: (i + GRID, 0))
    return pl.pallas_call(
        _combine_body,
        grid=(GRID,),
        in_specs=[f_spec0, f_spec1, f_spec0, f_spec1, f_spec0, f_spec0],
        out_specs=pl.BlockSpec((ROWS, D), lambda i: (i, 0)),
        out_shape=jax.ShapeDtypeStruct((N_PAD, D), jnp.float32),
    )(agg_lo, agg_lo, agg_hi, agg_hi, hf_lo, hf_hi)


def _head_body(x_ref, w_ref, o_ref):
    y = jnp.dot(x_ref[...], w_ref[...], preferred_element_type=jnp.float32)
    m = jnp.max(y, axis=-1, keepdims=True)
    ey = jnp.exp(y - m)
    o_ref[...] = ey / jnp.sum(ey, axis=-1, keepdims=True)


def _head(x_pad, W_out):
    return pl.pallas_call(
        _head_body,
        grid=(GRID,),
        in_specs=[pl.BlockSpec((ROWS, D), lambda i: (i, 0)),
                  pl.BlockSpec((D, D_OUT), lambda i: (0, 0))],
        out_specs=pl.BlockSpec((ROWS, D_OUT), lambda i: (i, 0)),
        out_shape=jax.ShapeDtypeStruct((N_PAD, D_OUT), jnp.float32),
    )(x_pad, W_out)


def _edot_block(cs, cd, ebig, off, mbuf):
    lanes = lax.iota(jnp.int32, 16)
    for g in range(B_E // 16):
        evec = jnp.zeros((16,), jnp.float32)
        for u in range(16):
            i = g * 16 + u
            acc = [jnp.zeros((16,), jnp.float32) for _ in range(4)]
            for j in range(DC // 32):
                va = plsc.bitcast(cs[i, pl.ds(16 * j, 16)], jnp.bfloat16)
                vb = plsc.bitcast(cd[i, pl.ds(16 * j, 16)], jnp.bfloat16)
                ps = va * vb
                x0, x1 = plsc.unpack(ps, format=plsc.PackFormat.INTERLEAVED)
                w = (2 * j) % 4
                acc[w] = acc[w] + x0
                acc[w + 1] = acc[w + 1] + x1
            tot = (acc[0] + acc[1]) + (acc[2] + acc[3])
            evec = jnp.where(lanes == u, jnp.sum(tot), evec)
        ebig[pl.ds(off + g * 16, 16)] = evec
        mbuf[...] = jnp.maximum(mbuf[...], evec)


def _epass(c_tab, src, dst):
    @functools.partial(
        pl.kernel,
        out_type=(jax.ShapeDtypeStruct((E_PAD,), jnp.float32),
                  jax.ShapeDtypeStruct((2 * (N_PAD // 128), 128), jnp.float32),
                  jax.ShapeDtypeStruct((2, 16), jnp.float32)),
        mesh=_MESH,
        compiler_params=_SC_PARAMS,
        scratch_types=[
            pltpu.VMEM((B_E,), jnp.int32),
            pltpu.VMEM((B_E,), jnp.int32),
            pltpu.VMEM((B_E,), jnp.int32),
            pltpu.VMEM((B_E,), jnp.int32),
            pltpu.VMEM((B_E, DC // 2), jnp.int32),
            pltpu.VMEM((B_E, DC // 2), jnp.int32),
            pltpu.VMEM((B_E, DC // 2), jnp.int32),
            pltpu.VMEM((B_E, DC // 2), jnp.int32),
            pltpu.VMEM((EPW,), jnp.float32),
            pltpu.VMEM((EPW,), jnp.int32),
            pltpu.VMEM((N_PAD // 128, 128), jnp.float32),
            pltpu.VMEM((N_PAD // 128,), jnp.int32),
            pltpu.VMEM((16,), jnp.float32),
            pltpu.VMEM((256,), jnp.float32),
            pltpu.VMEM_SHARED((N_PAD // 128, 128), jnp.float32),
            pltpu.VMEM_SHARED((256,), jnp.float32),
            pltpu.SemaphoreType.DMA,
            pltpu.SemaphoreType.DMA,
            pltpu.SemaphoreType.DMA,
            pltpu.SemaphoreType.DMA,
        ],
    )
    def k(c_hbm, src_hbm, dst_hbm, e_hbm, t_hbm, m_hbm,
          sidx0, didx0, sidx1, didx1, cs0, cd0, cs1, cd1,
          ebig, dbig, tbuf, ridx, mbuf, mall, tsh, msh,
          isem0, isem1, gsem0, gsem1):
        cid = lax.axis_index("c")
        sid = lax.axis_index("s")
        wid = cid * 16 + sid
        base = wid * EPW
        lanes = lax.iota(jnp.int32, 16)
        zero = jnp.zeros((16,), jnp.float32)

        mbuf[...] = zero - 3e38

        def idx_copy(b, sidx, didx, isem):
            pltpu.async_copy(src_hbm.at[pl.ds(base + b * B_E, B_E)], sidx, isem)
            pltpu.async_copy(dst_hbm.at[pl.ds(base + b * B_E, B_E)], didx, isem)

        def idx_wait(sidx, didx, isem):
            pltpu.make_async_copy(src_hbm.at[pl.ds(base, B_E)], sidx, isem).wait()
            pltpu.make_async_copy(dst_hbm.at[pl.ds(base, B_E)], didx, isem).wait()

        def gather(sidx, didx, cs, cd, gsem):
            pltpu.async_copy(c_hbm.at[sidx], cs, gsem)
            pltpu.async_copy(c_hbm.at[didx], cd, gsem)

        def gather_wait(cs, cd, gsem):
            pltpu.make_async_copy(c_hbm.at[pl.ds(0, B_E)], cs, gsem).wait()
            pltpu.make_async_copy(c_hbm.at[pl.ds(0, B_E)], cd, gsem).wait()

        idx_copy(0, sidx0, didx0, isem0)
        idx_wait(sidx0, didx0, isem0)
        gather(sidx0, didx0, cs0, cd0, gsem0)
        idx_copy(1, sidx1, didx1, isem1)

        @pl.loop(0, NBLK_E // 2)
        def _(t):
            b0 = 2 * t
            not_last = t < NBLK_E // 2 - 1

            idx_wait(sidx1, didx1, isem1)
            gather(sidx1, didx1, cs1, cd1, gsem1)
            gather_wait(cs0, cd0, gsem0)

            @pl.when(not_last)
            def _():
                idx_copy(b0 + 2, sidx0, didx0, isem0)
            _edot_block(cs0, cd0, ebig, b0 * B_E, mbuf)

            @pl.when(not_last)
            def _():
                idx_wait(sidx0, didx0, isem0)
                gather(sidx0, didx0, cs0, cd0, gsem0)
            gather_wait(cs1, cd1, gsem1)

            @pl.when(not_last)
            def _():
                idx_copy(b0 + 3, sidx1, didx1, isem1)
            _edot_block(cs1, cd1, ebig, (b0 + 1) * B_E, mbuf)

        @pl.loop(0, N_PAD // 128)
        def _(i):
            for j in range(8):
                tbuf[i, pl.ds(j * 16, 16)] = zero

        @pl.loop(0, N_PAD // 128 // 16)
        def _(i):
            ridx[pl.ds(i * 16, 16)] = lanes + i * 16

        pltpu.sync_copy(mbuf, msh.at[pl.ds(sid * 16, 16)])

        @pl.when(sid == 0)
        def _():
            pltpu.sync_copy(tbuf, tsh)
        plsc.subcore_barrier()

        pltpu.sync_copy(msh, mall)
        mv = jnp.maximum(mall[pl.ds(0, 16)], mall[pl.ds(16, 16)])
        for kk in range(2, 16):
            mv = jnp.maximum(mv, mall[pl.ds(kk * 16, 16)])
        mcv = jnp.where(lanes >= 0, jnp.max(mv), mv)

        pltpu.sync_copy(dst_hbm.at[pl.ds(base, EPW)], dbig)

        @pl.loop(0, EPW // 16)
        def _(i):
            dv = dbig[pl.ds(i * 16, 16)]
            ex = jnp.exp(ebig[pl.ds(i * 16, 16)] - mcv)
            plsc.addupdate_scatter(
                tbuf, [lax.shift_right_logical(dv, 7), lax.bitwise_and(dv, 127)], ex)

        pltpu.sync_copy(ebig, e_hbm.at[pl.ds(base, EPW)])
        pltpu.sync_copy(tbuf, tsh.at[ridx], add=True)

        @pl.when(sid == 0)
        def _():
            mbuf[...] = mcv
            pltpu.sync_copy(mbuf, m_hbm.at[cid])
        plsc.subcore_barrier()

        @pl.when(sid < 10)
        def _():
            pltpu.sync_copy(tsh.at[pl.ds(sid * 8, 8)],
                            t_hbm.at[pl.ds(cid * 80 + sid * 8, 8)])

    return k(c_tab, src, dst)


def _aggpass(hf_lo, hf_hi, src, dst, e, t_flat, m2, zslice):
    out_t = jax.ShapeDtypeStruct((2 * N_PAD, FCH), jnp.float32)

    @functools.partial(
        pl.kernel,
        out_type=(out_t, out_t),
        mesh=_MESH,
        compiler_params=_SC_PARAMS,
        scratch_types=[
            pltpu.VMEM((N_PAD // 128, 128), jnp.float32),
            pltpu.VMEM((16,), jnp.float32),
            pltpu.VMEM((16,), jnp.float32),
            pltpu.VMEM((B_A,), jnp.int32),
            pltpu.VMEM((B_A,), jnp.int32),
            pltpu.VMEM((B_A,), jnp.float32),
            pltpu.VMEM((B_A,), jnp.float32),
            pltpu.VMEM((B_A,), jnp.int32),
            pltpu.VMEM((B_A,), jnp.int32),
            pltpu.VMEM((B_A,), jnp.float32),
            pltpu.VMEM((B_A,), jnp.float32),
            pltpu.VMEM((B_A, FCH), jnp.float32),
            pltpu.VMEM((B_A, FCH), jnp.float32),
            pltpu.VMEM_SHARED((N_PAD, FCH), jnp.float32),
            pltpu.SemaphoreType.DMA,
            pltpu.SemaphoreType.DMA,
            pltpu.SemaphoreType.DMA,
            pltpu.SemaphoreType.DMA,
            pltpu.SemaphoreType.DMA,
            pltpu.SemaphoreType.DMA,
        ],
    )
    def k(hf_lo_hbm, hf_hi_hbm, src_hbm, dst_hbm, e_hbm, t_hbm, m_hbm, z_hbm,
          out_lo_hbm, out_hi_hbm, rtab, mb0, mb1,
          sidx0, didx0, ebuf0, abuf0, sidx1, didx1, ebuf1, abuf1,
          rows0, rows1, acc, isem0, isem1, gsem0, gsem1, ssem0, ssem1):
        cid = lax.axis_index("c")
        sid = lax.axis_index("s")
        wid = cid * 16 + sid
        base = wid * EPW
        my_row0 = sid * NSL
        nblk = EPW // B_A

        pltpu.sync_copy(t_hbm.at[pl.ds(0, N_PAD // 128)], rtab)
        pltpu.sync_copy(t_hbm.at[pl.ds(N_PAD // 128, N_PAD // 128)],
                        rows0.at[pl.ds(0, N_PAD // 128)])
        pltpu.sync_copy(m_hbm.at[0], mb0)
        pltpu.sync_copy(m_hbm.at[1], mb1)
        m0 = mb0[...]
        m1 = mb1[...]
        mx = jnp.maximum(m0, m1)
        w0 = jnp.exp(m0 - mx)
        w1 = jnp.exp(m1 - mx)

        @pl.loop(0, N_PAD // 128)
        def _(r):
            for j in range(8):
                sl = pl.ds(j * 16, 16)
                den = jnp.maximum(rtab[r, sl] * w0 + rows0[r, sl] * w1, 1e-37)
                rtab[r, sl] = 1.0 / den

        def idx_copy(b, sidx, didx, ebuf, isem):
            pltpu.async_copy(src_hbm.at[pl.ds(base + b * B_A, B_A)], sidx, isem)
            pltpu.async_copy(dst_hbm.at[pl.ds(base + b * B_A, B_A)], didx, isem)
            pltpu.async_copy(e_hbm.at[pl.ds(base + b * B_A, B_A)], ebuf, isem)

        def idx_wait(sidx, didx, ebuf, isem):
            pltpu.make_async_copy(src_hbm.at[pl.ds(base, B_A)], sidx, isem).wait()
            pltpu.make_async_copy(dst_hbm.at[pl.ds(base, B_A)], didx, isem).wait()
            pltpu.make_async_copy(e_hbm.at[pl.ds(base, B_A)], ebuf, isem).wait()

        def scale(hf_hbm, rows, didx, ebuf, abuf, gsem):
            pltpu.make_async_copy(hf_hbm.at[pl.ds(0, B_A)], rows, gsem).wait()

            @pl.loop(0, B_A // 16)
            def _(q):
                dv = didx[pl.ds(q * 16, 16)]
                dhi = lax.shift_right_logical(dv, 7)
                dlo = lax.bitwise_and(dv, 127)
                rv = plsc.load_gather(rtab, [dhi, dlo])
                abuf[pl.ds(q * 16, 16)] = (
                    jnp.exp(ebuf[pl.ds(q * 16, 16)] - mx) * rv)

            @pl.loop(0, B_A)
            def _(i):
                av = plsc.load_gather(abuf, [jnp.full((16,), i, jnp.int32)])
                for j in range(FCH // 16):
                    sl = pl.ds(j * 16, 16)
                    rows[i, sl] = rows[i, sl] * av

        def scat_wait(rows, ssem):
            pltpu.make_async_copy(rows, acc.at[pl.ds(0, B_A)], ssem).wait()

        for hf_hbm, out_hbm in ((hf_lo_hbm, out_lo_hbm), (hf_hi_hbm, out_hi_hbm)):
            pltpu.sync_copy(z_hbm.at[pl.ds(my_row0, NSL)],
                            acc.at[pl.ds(my_row0, NSL)])
            plsc.subcore_barrier()

            idx_copy(0, sidx0, didx0, ebuf0, isem0)
            idx_wait(sidx0, didx0, ebuf0, isem0)
            pltpu.async_copy(hf_hbm.at[sidx0], rows0, gsem0)
            idx_copy(1, sidx1, didx1, ebuf1, isem1)

            @pl.loop(0, nblk // 2)
            def _(t):
                b0 = 2 * t
                not_last = t < nblk // 2 - 1

                idx_wait(sidx1, didx1, ebuf1, isem1)
                pltpu.async_copy(hf_hbm.at[sidx1], rows1, gsem1)
                scale(hf_hbm, rows0, didx0, ebuf0, abuf0, gsem0)
                pltpu.async_copy(rows0, acc.at[didx0], ssem0, add=True)

                scale(hf_hbm, rows1, didx1, ebuf1, abuf1, gsem1)
                pltpu.async_copy(rows1, acc.at[didx1], ssem1, add=True)

                @pl.when(not_last)
                def _():
                    scat_wait(rows0, ssem0)
                    idx_copy(b0 + 2, sidx0, didx0, ebuf0, isem0)
                    idx_wait(sidx0, didx0, ebuf0, isem0)
                    pltpu.async_copy(hf_hbm.at[sidx0], rows0, gsem0)
                    scat_wait(rows1, ssem1)
                    idx_copy(b0 + 3, sidx1, didx1, ebuf1, isem1)

            scat_wait(rows0, ssem0)
            scat_wait(rows1, ssem1)
            plsc.subcore_barrier()
            pltpu.sync_copy(acc.at[pl.ds(my_row0, NSL)],
                            out_hbm.at[pl.ds(cid * N_PAD + my_row0, NSL)])
            plsc.subcore_barrier()

    return k(hf_lo, hf_hi, src, dst, e, t_flat, m2, zslice)


def _layer(h_pad, src_p, dst_p, zslice, W_fc, W_mu, b_mu, W_ls, b_ls):
    hf_lo, hf_hi, c = _stage1(h_pad, W_fc, W_mu, b_mu, W_ls, b_ls)
    c_i32 = lax.bitcast_convert_type(c.reshape(N_PAD, DC // 2, 2), jnp.int32)
    e, t_flat, m2 = _epass(c_i32, src_p, dst_p)
    agg_lo, agg_hi = _aggpass(hf_lo, hf_hi, src_p, dst_p, e, t_flat, m2, zslice)
    return _combine(agg_lo, agg_hi, hf_lo, hf_hi)


def kernel(h, edge_index, W_fc0, W_mu0, b_mu0, W_ls0, b_ls0,
           W_fc1, W_mu1, b_mu1, W_ls1, b_ls1, W_out):
    src_p = jnp.concatenate([edge_index[0], jnp.zeros((E_PAD - E,), jnp.int32)])
    dst_p = jnp.concatenate(
        [edge_index[1], jnp.full((E_PAD - E,), N_PAD - 1, jnp.int32)])
    zslice = jnp.zeros((N_PAD, FCH), jnp.float32)
    h_pad = jnp.pad(h, ((0, N_PAD - N), (0, 0)))
    h1 = _layer(h_pad, src_p, dst_p, zslice, W_fc0, W_mu0, b_mu0, W_ls0, b_ls0)
    h2 = _layer(h1, src_p, dst_p, zslice, W_fc1, W_mu1, b_mu1, W_ls1, b_ls1)
    return _head(h2, W_out)[:N]
